# Initial kernel scaffold; baseline (speedup 1.0000x reference)
import functools

import jax
import jax.numpy as jnp
from jax import lax
from jax.experimental import pallas as pl
from jax.experimental.pallas import tpu as pltpu
from jax.experimental.pallas import tpu_sc as plsc

N = 100000
D = 128
ROWS_PER_DEG = 16000
NUM_WORKERS = 32
ROWS_PER_WORKER = ROWS_PER_DEG // NUM_WORKERS
HALF = ROWS_PER_WORKER // 2
PADH = 256
NUM_COLS = 21
_OFF = (0, 1, 3, 6, 10, 15)


def _acc_add(acc, buf):
    def body(r, carry):
        for cc in range(D // 16):
            sl = pl.ds(cc * 16, 16)
            plsc.addupdate(acc.at[r, sl], buf[r, sl])
        return carry
    lax.fori_loop(0, PADH, body, 0, unroll=2)


def _sc_gather_sum(nf, idx):
    mesh = plsc.VectorSubcoreMesh(core_axis_name="c", subcore_axis_name="s")

    @functools.partial(
        pl.kernel,
        out_type=jax.ShapeDtypeStruct((2 * 6 * NUM_WORKERS, HALF, D),
                                      jnp.float32),
        mesh=mesh,
        scratch_types=[
            pltpu.VMEM((2 * NUM_COLS, PADH), jnp.int32),
            pltpu.VMEM((PADH, D), jnp.float32),
            pltpu.VMEM((PADH, D), jnp.float32),
            pltpu.VMEM((PADH, D), jnp.float32),
            pltpu.SemaphoreType.DMA,
            pltpu.SemaphoreType.DMA,
            pltpu.SemaphoreType.DMA,
        ],
    )
    def k(nf_hbm, idx_hbm, out_hbm, idx_v, acc, buf0, buf1,
          sem_a, sem_b0, sem_b1):
        cid = lax.axis_index("c")
        sid = lax.axis_index("s")
        wid = sid * 2 + cid
        pltpu.sync_copy(idx_hbm.at[wid], idx_v)
        bufs = (buf0, buf1)
        sems = (sem_b0, sem_b1)
        for d in range(1, 7):
            c0 = _OFF[d - 1]
            for h in range(2):
                cp_acc = pltpu.async_copy(
                    nf_hbm.at[idx_v.at[2 * c0 + h]], acc, sem_a)
                cps = [None, None]
                if d > 1:
                    cps[1] = pltpu.async_copy(
                        nf_hbm.at[idx_v.at[2 * (c0 + 1) + h]], bufs[1],
                        sems[1])
                cp_acc.wait()
                for j in range(1, d):
                    bj = j % 2
                    cps[bj].wait()
                    if j + 1 < d:
                        bn = (j + 1) % 2
                        cps[bn] = pltpu.async_copy(
                            nf_hbm.at[idx_v.at[2 * (c0 + j + 1) + h]],
                            bufs[bn], sems[bn])
                    _acc_add(acc, bufs[bj])
                blk = ((d - 1) * NUM_WORKERS + wid) * 2 + h
                pltpu.sync_copy(acc.at[pl.ds(0, HALF)], out_hbm.at[blk])

    return k(nf, idx)


def _tc_linear(nf, nsum, W, b):
    BS = 4000
    nblocks = N // BS

    def ws_idx(g):
        return (jnp.where(g == 0, 0, 2 * ((g + 3) // 4)), 0, 0)

    def wn_idx(g):
        return (jnp.maximum(2 * ((g + 3) // 4) - 1, 0), 0, 0)

    def body(x_ref, ns_ref, ws_ref, wn_ref, bs_ref, bn_ref, o_ref):
        g = pl.program_id(0)
        o_ref[...] = jnp.dot(
            x_ref[...], ws_ref[0], preferred_element_type=jnp.float32,
            precision=lax.Precision.HIGHEST) + bs_ref[0, 0]

        @pl.when(g > 0)
        def _():
            o_ref[...] += jnp.dot(
                ns_ref[...], wn_ref[0], preferred_element_type=jnp.float32,
                precision=lax.Precision.HIGHEST) + bn_ref[0, 0]

    br = b.reshape(b.shape[0], 1, D)
    return pl.pallas_call(
        body,
        grid=(nblocks,),
        in_specs=[
            pl.BlockSpec((BS, D), lambda g: (g, 0)),
            pl.BlockSpec((BS, D), lambda g: (jnp.maximum(g - 1, 0), 0)),
            pl.BlockSpec((1, D, D), ws_idx),
            pl.BlockSpec((1, D, D), wn_idx),
            pl.BlockSpec((1, 1, D), ws_idx),
            pl.BlockSpec((1, 1, D), wn_idx),
        ],
        out_specs=pl.BlockSpec((BS, D), lambda g: (g, 0)),
        out_shape=jax.ShapeDtypeStruct((N, D), jnp.float32),
    )(nf, nsum, W, W, br, br)


def kernel(node_features, deg_slice, deg_adj_1, deg_adj_2, deg_adj_3,
           deg_adj_4, deg_adj_5, deg_adj_6, W, b):
    adjs = (deg_adj_1, deg_adj_2, deg_adj_3, deg_adj_4, deg_adj_5, deg_adj_6)
    cols = jnp.concatenate([a.astype(jnp.int32).T for a in adjs], axis=0)
    idx = cols.reshape(NUM_COLS, NUM_WORKERS, 2, HALF)
    idx = jnp.pad(idx, ((0, 0), (0, 0), (0, 0), (0, PADH - HALF)))
    idx = idx.transpose(1, 0, 2, 3).reshape(NUM_WORKERS, 2 * NUM_COLS, PADH)
    nsum = _sc_gather_sum(node_features, idx)
    nsum = nsum.reshape(6 * ROWS_PER_DEG, D)
    return _tc_linear(node_features, nsum, W, b)


# speedup vs baseline: 1.1684x; 1.1684x over previous
import functools

import jax
import jax.numpy as jnp
from jax import lax
from jax.experimental import pallas as pl
from jax.experimental.pallas import tpu as pltpu
from jax.experimental.pallas import tpu_sc as plsc

N = 100000
D = 128
ROWS_PER_DEG = 16000
NUM_WORKERS = 32
NQ = 4
CHUNK = 125
PADC = 128
NUM_COLS = 21
_OFF = (0, 1, 3, 6, 10, 15)


def _acc_add(acc, buf):
    def body(r, carry):
        for cc in range(D // 16):
            sl = pl.ds(cc * 16, 16)
            plsc.addupdate(acc.at[r, sl], buf[r, sl])
        return carry
    lax.fori_loop(0, PADC, body, 0, unroll=2)


def _sc_gather_sum(nf, idx):
    mesh = plsc.VectorSubcoreMesh(core_axis_name="c", subcore_axis_name="s")

    @functools.partial(
        pl.kernel,
        out_type=jax.ShapeDtypeStruct((NQ * 6 * NUM_WORKERS, CHUNK, D),
                                      jnp.float32),
        mesh=mesh,
        scratch_types=[
            pltpu.VMEM((NQ * NUM_COLS, PADC), jnp.int32),
            pltpu.VMEM((PADC, D), jnp.float32),
            pltpu.VMEM((PADC, D), jnp.float32),
            pltpu.VMEM((PADC, D), jnp.float32),
            pltpu.SemaphoreType.DMA,
            pltpu.SemaphoreType.DMA,
            pltpu.SemaphoreType.DMA,
        ],
    )
    def k(nf_hbm, idx_hbm, out_hbm, idx_v, acc, buf0, buf1,
          sem_a, sem_b0, sem_b1):
        cid = lax.axis_index("c")
        sid = lax.axis_index("s")
        wid = sid * 2 + cid
        pltpu.sync_copy(idx_hbm.at[wid], idx_v)
        bufs = (buf0, buf1)
        sems = (sem_b0, sem_b1)
        for d in range(1, 7):
            c0 = _OFF[d - 1]
            for q in range(NQ):
                cp_acc = pltpu.async_copy(
                    nf_hbm.at[idx_v.at[NQ * c0 + q]], acc, sem_a)
                cps = [None, None]
                if d > 1:
                    cps[1] = pltpu.async_copy(
                        nf_hbm.at[idx_v.at[NQ * (c0 + 1) + q]], bufs[1],
                        sems[1])
                cp_acc.wait()
                for j in range(1, d):
                    bj = j % 2
                    cps[bj].wait()
                    if j + 1 < d:
                        bn = (j + 1) % 2
                        cps[bn] = pltpu.async_copy(
                            nf_hbm.at[idx_v.at[NQ * (c0 + j + 1) + q]],
                            bufs[bn], sems[bn])
                    _acc_add(acc, bufs[bj])
                blk = ((d - 1) * NUM_WORKERS + wid) * NQ + q
                pltpu.sync_copy(acc.at[pl.ds(0, CHUNK)], out_hbm.at[blk])

    return k(nf, idx)


def _tc_linear(nf, nsum, W, b):
    BS = 4000
    nblocks = N // BS

    def ws_idx(g):
        return (jnp.where(g == 0, 0, 2 * ((g + 3) // 4)), 0, 0)

    def wn_idx(g):
        return (jnp.maximum(2 * ((g + 3) // 4) - 1, 0), 0, 0)

    def body(x_ref, ns_ref, ws_ref, wn_ref, bs_ref, bn_ref, o_ref):
        g = pl.program_id(0)
        o_ref[...] = jnp.dot(
            x_ref[...], ws_ref[0], preferred_element_type=jnp.float32,
            precision=lax.Precision.HIGHEST) + bs_ref[0, 0]

        @pl.when(g > 0)
        def _():
            o_ref[...] += jnp.dot(
                ns_ref[...], wn_ref[0], preferred_element_type=jnp.float32,
                precision=lax.Precision.HIGHEST) + bn_ref[0, 0]

    br = b.reshape(b.shape[0], 1, D)
    return pl.pallas_call(
        body,
        grid=(nblocks,),
        in_specs=[
            pl.BlockSpec((BS, D), lambda g: (g, 0)),
            pl.BlockSpec((BS, D), lambda g: (jnp.maximum(g - 1, 0), 0)),
            pl.BlockSpec((1, D, D), ws_idx),
            pl.BlockSpec((1, D, D), wn_idx),
            pl.BlockSpec((1, 1, D), ws_idx),
            pl.BlockSpec((1, 1, D), wn_idx),
        ],
        out_specs=pl.BlockSpec((BS, D), lambda g: (g, 0)),
        out_shape=jax.ShapeDtypeStruct((N, D), jnp.float32),
    )(nf, nsum, W, W, br, br)


def kernel(node_features, deg_slice, deg_adj_1, deg_adj_2, deg_adj_3,
           deg_adj_4, deg_adj_5, deg_adj_6, W, b):
    adjs = (deg_adj_1, deg_adj_2, deg_adj_3, deg_adj_4, deg_adj_5, deg_adj_6)
    cols = jnp.concatenate([a.astype(jnp.int32).T for a in adjs], axis=0)
    idx = cols.reshape(NUM_COLS, NUM_WORKERS, NQ, CHUNK)
    idx = jnp.pad(idx, ((0, 0), (0, 0), (0, 0), (0, PADC - CHUNK)))
    idx = idx.transpose(1, 0, 2, 3).reshape(NUM_WORKERS, NQ * NUM_COLS, PADC)
    nsum = _sc_gather_sum(node_features, idx)
    nsum = nsum.reshape(6 * ROWS_PER_DEG, D)
    return _tc_linear(node_features, nsum, W, b)
